# Initial kernel scaffold; baseline (speedup 1.0000x reference)
import jax
import jax.numpy as jnp
from jax.experimental import pallas as pl


def kernel(x, edge_index, edge_type, weight, w_comp, loop_weight, h_bias):
    raise NotImplementedError("write your pallas kernel here")


# speedup vs baseline: 10.1360x; 10.1360x over previous
import jax
import jax.numpy as jnp
from jax import lax
from jax.experimental import pallas as pl
from jax.experimental.pallas import tpu as pltpu
from jax.experimental.pallas import tpu_sc as plsc

_N = 10000
_E = 320000
_R = 8
_B = 4
_D = 128
_C = 16
_NCHUNK = _D // _C
_RN = _R * _N

_NCORE = 2
_NSUB = 16
_PASSES = _NCHUNK // _NCORE
_EPT = _E // _NSUB
_W = 80
_NWIN = _EPT // _W
_NB = 5
_RPT = _RN // _NSUB
_ZROWS = 40


def _sc_body(xt, srch, dsth, typh, aggo, cnto,
             agg_s, buf_r, gidx, gbs, zb3, ones_v, sems):
    core = lax.axis_index("c")
    sub = lax.axis_index("s")

    pltpu.sync_copy(dsth.at[sub], buf_r)
    pltpu.sync_copy(typh.at[sub], gidx)

    def _mk_ridx(w, carry):
        for j in range(_W // 16):
            t16 = gidx[w, pl.ds(j * 16, 16)]
            d16 = buf_r[w, pl.ds(j * 16, 16)]
            buf_r[w, pl.ds(j * 16, 16)] = t16 * _N + d16
        return carry
    lax.fori_loop(0, _NWIN, _mk_ridx, 0)
    pltpu.sync_copy(srch.at[sub], gidx)

    z16 = jnp.zeros((16,), jnp.float32)

    def _fill_zb3(i, carry):
        zb3[i, :] = z16
        return carry
    lax.fori_loop(0, _ZROWS, _fill_zb3, 0)

    one16 = jnp.ones((16,), jnp.float32)

    def _fill_ones(i, carry):
        ones_v[i, :] = one16
        return carry
    lax.fori_loop(0, _W, _fill_ones, 0)

    def _zero_agg():
        def _z(z, c2):
            pltpu.sync_copy(zb3, agg_s.at[pl.ds(sub * _RPT + z * _ZROWS, _ZROWS)])
            return c2
        lax.fori_loop(0, _RPT // _ZROWS, _z, 0)

    _zero_agg()
    plsc.subcore_barrier()
    halfw = _NWIN // 2
    w0 = core * halfw

    def _cnt_win(w, carry):
        pltpu.sync_copy(ones_v, agg_s.at[buf_r.at[w0 + w]], add=True)
        return carry
    lax.fori_loop(0, halfw, _cnt_win, 0)
    plsc.subcore_barrier()
    pltpu.sync_copy(agg_s.at[pl.ds(sub * _RPT, _RPT)],
                    cnto.at[core, pl.ds(sub * _RPT, _RPT), :])

    for p in range(_PASSES):
        off = core * _PASSES * _N if p == 0 else _N

        def _mk_gidx(w, carry):
            for j in range(_W // 16):
                s16 = gidx[w, pl.ds(j * 16, 16)]
                gidx[w, pl.ds(j * 16, 16)] = s16 + off
            return carry
        lax.fori_loop(0, _NWIN, _mk_gidx, 0)

        _zero_agg()
        plsc.subcore_barrier()

        def _group(i, carry):
            wbase = i * _NB
            descs = [
                pltpu.async_copy(xt.at[gidx.at[wbase + b]], gbs.at[b], sems.at[b])
                for b in range(_NB)
            ]
            for b in range(_NB):
                descs[b].wait()
                pltpu.sync_copy(gbs.at[b], agg_s.at[buf_r.at[wbase + b]], add=True)
            return carry
        lax.fori_loop(0, _NWIN // _NB, _group, 0)
        plsc.subcore_barrier()

        pltpu.sync_copy(
            agg_s.at[pl.ds(sub * _RPT, _RPT)],
            aggo.at[core * _PASSES + p, pl.ds(sub * _RPT, _RPT), :])


_sc_call = pl.kernel(
    _sc_body,
    out_type=(
        jax.ShapeDtypeStruct((_NCHUNK, _RN, _C), jnp.float32),
        jax.ShapeDtypeStruct((_NCORE, _RN, _C), jnp.float32),
    ),
    mesh=plsc.VectorSubcoreMesh(core_axis_name="c", subcore_axis_name="s"),
    compiler_params=pltpu.CompilerParams(use_tc_tiling_on_sc=False),
    scratch_types=[
        pltpu.VMEM_SHARED((_RN, _C), jnp.float32),
        pltpu.VMEM((_NWIN, _W), jnp.int32),
        pltpu.VMEM((_NWIN, _W), jnp.int32),
        pltpu.VMEM((_NB, _W, _C), jnp.float32),
        pltpu.VMEM((_ZROWS, _C), jnp.float32),
        pltpu.VMEM((_W, _C), jnp.float32),
        pltpu.SemaphoreType.DMA((_NB,)),
    ],
)

_BN = 1000


def _tc_body(x_ref, agg_ref, cnt_ref, w_ref, wc_ref, lw_ref, b_ref, o_ref):
    cnt = cnt_ref[0, :, :, 0] + cnt_ref[1, :, :, 0]
    recip = 1.0 / jnp.maximum(cnt, 1.0)
    acc = jnp.dot(x_ref[...], lw_ref[...], preferred_element_type=jnp.float32)
    g = [None] * _B
    for r in range(_R):
        a = agg_ref[r] * recip[r][:, None]
        for b in range(_B):
            contrib = wc_ref[r, b] * a
            g[b] = contrib if g[b] is None else g[b] + contrib
    for b in range(_B):
        acc = acc + jnp.dot(g[b], w_ref[b], preferred_element_type=jnp.float32)
    o_ref[...] = acc + b_ref[...]


_tc_call = pl.pallas_call(
    _tc_body,
    grid=(_N // _BN,),
    in_specs=[
        pl.BlockSpec((_BN, _D), lambda i: (i, 0)),
        pl.BlockSpec((_R, _BN, _D), lambda i: (0, i, 0)),
        pl.BlockSpec((_NCORE, _R, _BN, _C), lambda i: (0, 0, i, 0)),
        pl.BlockSpec((_B, _D, _D), lambda i: (0, 0, 0)),
        pl.BlockSpec(memory_space=pltpu.SMEM),
        pl.BlockSpec((_D, _D), lambda i: (0, 0)),
        pl.BlockSpec((1, _D), lambda i: (0, 0)),
    ],
    out_specs=pl.BlockSpec((_BN, _D), lambda i: (i, 0)),
    out_shape=jax.ShapeDtypeStruct((_N, _D), jnp.float32),
)


def kernel(x, edge_index, edge_type, weight, w_comp, loop_weight, h_bias):
    src = edge_index[0].reshape(_NSUB, _NWIN, _W)
    dst = edge_index[1].reshape(_NSUB, _NWIN, _W)
    typ = edge_type.reshape(_NSUB, _NWIN, _W)
    xt = x.reshape(_N, _NCHUNK, _C).transpose(1, 0, 2).reshape(_NCHUNK * _N, _C)
    agg, cnt = _sc_call(xt, src, dst, typ)
    aggv = agg.reshape(_NCHUNK, _R, _N, _C).transpose(1, 2, 0, 3).reshape(_R, _N, _D)
    cntv = cnt.reshape(_NCORE, _R, _N, _C)
    return _tc_call(x, aggv, cntv, weight, w_comp, loop_weight,
                    h_bias.reshape(1, _D))


# speedup vs baseline: 13.1802x; 1.3003x over previous
import jax
import jax.numpy as jnp
from jax import lax
from jax.experimental import pallas as pl
from jax.experimental.pallas import tpu as pltpu
from jax.experimental.pallas import tpu_sc as plsc

_N = 10000
_E = 320000
_R = 8
_B = 4
_D = 128
_C = 16
_NCHUNK = _D // _C
_RN = _R * _N

_NCORE = 2
_NSUB = 16
_PASSES = _NCHUNK // _NCORE
_EPT = _E // _NSUB
_W = 80
_NWIN = _EPT // _W
_NB = 5
_RPT = _RN // _NSUB
_ZROWS = 40


def _sc_body(xt, srch, dsth, typh, aggo, cnto,
             agg_s, buf_r, gidx, gbs, zb3, ones_v, sems):
    core = lax.axis_index("c")
    sub = lax.axis_index("s")

    pltpu.sync_copy(dsth.at[sub], buf_r)
    pltpu.sync_copy(typh.at[sub], gidx)

    def _mk_ridx(w, carry):
        for j in range(_W // 16):
            t16 = gidx[w, pl.ds(j * 16, 16)]
            d16 = buf_r[w, pl.ds(j * 16, 16)]
            buf_r[w, pl.ds(j * 16, 16)] = t16 * _N + d16
        return carry
    lax.fori_loop(0, _NWIN, _mk_ridx, 0)
    pltpu.sync_copy(srch.at[sub], gidx)

    z16 = jnp.zeros((16,), jnp.float32)

    def _fill_zb3(i, carry):
        zb3[i, :] = z16
        return carry
    lax.fori_loop(0, _ZROWS, _fill_zb3, 0)

    one16 = jnp.ones((16,), jnp.float32)

    def _fill_ones(i, carry):
        ones_v[i, :] = one16
        return carry
    lax.fori_loop(0, _W, _fill_ones, 0)

    def _zero_agg():
        def _z(z, c2):
            pltpu.sync_copy(zb3, agg_s.at[pl.ds(sub * _RPT + z * _ZROWS, _ZROWS)])
            return c2
        lax.fori_loop(0, _RPT // _ZROWS, _z, 0)

    _zero_agg()
    plsc.subcore_barrier()
    halfw = _NWIN // 2
    w0 = core * halfw

    def _cnt_win(w, carry):
        pltpu.sync_copy(ones_v, agg_s.at[buf_r.at[w0 + w]], add=True)
        return carry
    lax.fori_loop(0, halfw, _cnt_win, 0)
    plsc.subcore_barrier()
    pltpu.sync_copy(agg_s.at[pl.ds(sub * _RPT, _RPT)],
                    cnto.at[core, pl.ds(sub * _RPT, _RPT), :])

    for p in range(_PASSES):
        off = core * _PASSES * _N if p == 0 else _N

        def _mk_gidx(w, carry):
            for j in range(_W // 16):
                s16 = gidx[w, pl.ds(j * 16, 16)]
                gidx[w, pl.ds(j * 16, 16)] = s16 + off
            return carry
        lax.fori_loop(0, _NWIN, _mk_gidx, 0)

        _zero_agg()
        plsc.subcore_barrier()

        def _group(i, carry):
            wbase = i * _NB
            descs = [
                pltpu.async_copy(xt.at[gidx.at[wbase + b]], gbs.at[b], sems.at[b])
                for b in range(_NB)
            ]
            for b in range(_NB):
                descs[b].wait()
                pltpu.sync_copy(gbs.at[b], agg_s.at[buf_r.at[wbase + b]], add=True)
            return carry
        lax.fori_loop(0, _NWIN // _NB, _group, 0)
        plsc.subcore_barrier()

        pltpu.sync_copy(
            agg_s.at[pl.ds(sub * _RPT, _RPT)],
            aggo.at[pl.ds(sub * _RPT, _RPT), pl.ds((core * _PASSES + p) * _C, _C)])


_sc_call = pl.kernel(
    _sc_body,
    out_type=(
        jax.ShapeDtypeStruct((_RN, _D), jnp.float32),
        jax.ShapeDtypeStruct((_NCORE, _RN, _C), jnp.float32),
    ),
    mesh=plsc.VectorSubcoreMesh(core_axis_name="c", subcore_axis_name="s"),
    compiler_params=pltpu.CompilerParams(use_tc_tiling_on_sc=False),
    scratch_types=[
        pltpu.VMEM_SHARED((_RN, _C), jnp.float32),
        pltpu.VMEM((_NWIN, _W), jnp.int32),
        pltpu.VMEM((_NWIN, _W), jnp.int32),
        pltpu.VMEM((_NB, _W, _C), jnp.float32),
        pltpu.VMEM((_ZROWS, _C), jnp.float32),
        pltpu.VMEM((_W, _C), jnp.float32),
        pltpu.SemaphoreType.DMA((_NB,)),
    ],
)

_BN = 1000


def _tc_body(x_ref, agg_ref, cnt_ref, w_ref, wc_ref, lw_ref, b_ref, o_ref):
    cnt = cnt_ref[0, :, :, 0] + cnt_ref[1, :, :, 0]
    recip = 1.0 / jnp.maximum(cnt, 1.0)
    acc = jnp.dot(x_ref[...], lw_ref[...], preferred_element_type=jnp.float32)
    g = [None] * _B
    for r in range(_R):
        a = agg_ref[r] * recip[r][:, None]
        for b in range(_B):
            contrib = wc_ref[r, b] * a
            g[b] = contrib if g[b] is None else g[b] + contrib
    for b in range(_B):
        acc = acc + jnp.dot(g[b], w_ref[b], preferred_element_type=jnp.float32)
    o_ref[...] = acc + b_ref[...]


_tc_call = pl.pallas_call(
    _tc_body,
    grid=(_N // _BN,),
    in_specs=[
        pl.BlockSpec((_BN, _D), lambda i: (i, 0)),
        pl.BlockSpec((_R, _BN, _D), lambda i: (0, i, 0)),
        pl.BlockSpec((_NCORE, _R, _BN, _C), lambda i: (0, 0, i, 0)),
        pl.BlockSpec((_B, _D, _D), lambda i: (0, 0, 0)),
        pl.BlockSpec(memory_space=pltpu.SMEM),
        pl.BlockSpec((_D, _D), lambda i: (0, 0)),
        pl.BlockSpec((1, _D), lambda i: (0, 0)),
    ],
    out_specs=pl.BlockSpec((_BN, _D), lambda i: (i, 0)),
    out_shape=jax.ShapeDtypeStruct((_N, _D), jnp.float32),
)


def kernel(x, edge_index, edge_type, weight, w_comp, loop_weight, h_bias):
    src = edge_index[0].reshape(_NSUB, _NWIN, _W)
    dst = edge_index[1].reshape(_NSUB, _NWIN, _W)
    typ = edge_type.reshape(_NSUB, _NWIN, _W)
    xt = x.reshape(_N, _NCHUNK, _C).transpose(1, 0, 2).reshape(_NCHUNK * _N, _C)
    agg, cnt = _sc_call(xt, src, dst, typ)
    aggv = agg.reshape(_R, _N, _D)
    cntv = cnt.reshape(_NCORE, _R, _N, _C)
    return _tc_call(x, aggv, cntv, weight, w_comp, loop_weight,
                    h_bias.reshape(1, _D))


# speedup vs baseline: 13.9244x; 1.0565x over previous
import jax
import jax.numpy as jnp
from jax import lax
from jax.experimental import pallas as pl
from jax.experimental.pallas import tpu as pltpu
from jax.experimental.pallas import tpu_sc as plsc

_N = 10000
_E = 320000
_R = 8
_B = 4
_D = 128
_C = 16
_NCHUNK = _D // _C
_RN = _R * _N

_NCORE = 2
_NSUB = 16
_PASSES = _NCHUNK // _NCORE
_W = 128
_NWIN = 160
_EPAD = _NSUB * _NWIN * _W
_NB = 4
_RPT = _RN // _NSUB
_ZROWS = 40


def _sc_body(xt, srch, dsth, typh, aggo, cnto,
             agg_s, buf_r, gidx, gbs, zb3, sems):
    core = lax.axis_index("c")
    sub = lax.axis_index("s")

    pltpu.sync_copy(dsth.at[sub], buf_r)
    pltpu.sync_copy(typh.at[sub], gidx)

    def _mk_ridx(w, carry):
        for j in range(_W // 16):
            t16 = gidx[w, pl.ds(j * 16, 16)]
            d16 = buf_r[w, pl.ds(j * 16, 16)]
            buf_r[w, pl.ds(j * 16, 16)] = t16 * _N + d16
        return carry
    lax.fori_loop(0, _NWIN, _mk_ridx, 0)
    pltpu.sync_copy(srch.at[sub], gidx)

    z16 = jnp.zeros((16,), jnp.float32)

    def _fill_zb3(i, carry):
        zb3[i, :] = z16
        return carry
    lax.fori_loop(0, _ZROWS, _fill_zb3, 0)

    one16 = jnp.ones((16,), jnp.float32)

    def _fill_ones(i, carry):
        gbs[0, i, :] = one16
        return carry
    lax.fori_loop(0, _W, _fill_ones, 0)

    def _zero_agg():
        def _z(z, c2):
            pltpu.sync_copy(zb3, agg_s.at[pl.ds(sub * _RPT + z * _ZROWS, _ZROWS)])
            return c2
        lax.fori_loop(0, _RPT // _ZROWS, _z, 0)

    _zero_agg()
    plsc.subcore_barrier()
    halfw = _NWIN // 2
    w0 = core * halfw

    def _cnt_win(w, carry):
        pltpu.sync_copy(gbs.at[0], agg_s.at[buf_r.at[w0 + w]], add=True)
        return carry
    lax.fori_loop(0, halfw, _cnt_win, 0)
    plsc.subcore_barrier()
    pltpu.sync_copy(agg_s.at[pl.ds(sub * _RPT, _RPT)],
                    cnto.at[core, pl.ds(sub * _RPT, _RPT), :])

    for p in range(_PASSES):
        off = core * _PASSES * _N if p == 0 else _N

        def _mk_gidx(w, carry):
            for j in range(_W // 16):
                s16 = gidx[w, pl.ds(j * 16, 16)]
                gidx[w, pl.ds(j * 16, 16)] = s16 + off
            return carry
        lax.fori_loop(0, _NWIN, _mk_gidx, 0)

        _zero_agg()
        plsc.subcore_barrier()

        def _group(i, carry):
            wbase = i * _NB
            descs = [
                pltpu.async_copy(xt.at[gidx.at[wbase + b]], gbs.at[b], sems.at[b])
                for b in range(_NB)
            ]
            for b in range(_NB):
                descs[b].wait()
                pltpu.sync_copy(gbs.at[b], agg_s.at[buf_r.at[wbase + b]], add=True)
            return carry
        lax.fori_loop(0, _NWIN // _NB, _group, 0)
        plsc.subcore_barrier()

        pltpu.sync_copy(
            agg_s.at[pl.ds(sub * _RPT, _RPT)],
            aggo.at[pl.ds(sub * _RPT, _RPT), pl.ds((core * _PASSES + p) * _C, _C)])


_sc_call = pl.kernel(
    _sc_body,
    out_type=(
        jax.ShapeDtypeStruct((_RN, _D), jnp.float32),
        jax.ShapeDtypeStruct((_NCORE, _RN, _C), jnp.float32),
    ),
    mesh=plsc.VectorSubcoreMesh(core_axis_name="c", subcore_axis_name="s"),
    compiler_params=pltpu.CompilerParams(use_tc_tiling_on_sc=False),
    scratch_types=[
        pltpu.VMEM_SHARED((_RN + _W, _C), jnp.float32),
        pltpu.VMEM((_NWIN, _W), jnp.int32),
        pltpu.VMEM((_NWIN, _W), jnp.int32),
        pltpu.VMEM((_NB, _W, _C), jnp.float32),
        pltpu.VMEM((_ZROWS, _C), jnp.float32),
        pltpu.SemaphoreType.DMA((_NB,)),
    ],
)

_BN = 1000


def _tc_body(x_ref, agg_ref, cnt_ref, w_ref, wc_ref, lw_ref, b_ref, o_ref):
    cnt = cnt_ref[0, :, :, 0] + cnt_ref[1, :, :, 0]
    recip = 1.0 / jnp.maximum(cnt, 1.0)
    acc = jnp.dot(x_ref[...], lw_ref[...], preferred_element_type=jnp.float32)
    g = [None] * _B
    for r in range(_R):
        a = agg_ref[r] * recip[r][:, None]
        for b in range(_B):
            contrib = wc_ref[r, b] * a
            g[b] = contrib if g[b] is None else g[b] + contrib
    for b in range(_B):
        acc = acc + jnp.dot(g[b], w_ref[b], preferred_element_type=jnp.float32)
    o_ref[...] = acc + b_ref[...]


_tc_call = pl.pallas_call(
    _tc_body,
    grid=(_N // _BN,),
    in_specs=[
        pl.BlockSpec((_BN, _D), lambda i: (i, 0)),
        pl.BlockSpec((_R, _BN, _D), lambda i: (0, i, 0)),
        pl.BlockSpec((_NCORE, _R, _BN, _C), lambda i: (0, 0, i, 0)),
        pl.BlockSpec((_B, _D, _D), lambda i: (0, 0, 0)),
        pl.BlockSpec(memory_space=pltpu.SMEM),
        pl.BlockSpec((_D, _D), lambda i: (0, 0)),
        pl.BlockSpec((1, _D), lambda i: (0, 0)),
    ],
    out_specs=pl.BlockSpec((_BN, _D), lambda i: (i, 0)),
    out_shape=jax.ShapeDtypeStruct((_N, _D), jnp.float32),
)


def kernel(x, edge_index, edge_type, weight, w_comp, loop_weight, h_bias):
    npad = _EPAD - _E
    pi = jnp.arange(npad, dtype=jnp.int32)
    src = jnp.concatenate([edge_index[0], pi % _N]).reshape(_NSUB, _NWIN, _W)
    dst = jnp.concatenate([edge_index[1], pi % _W]).reshape(_NSUB, _NWIN, _W)
    typ = jnp.concatenate(
        [edge_type, jnp.full((npad,), _R, jnp.int32)]).reshape(_NSUB, _NWIN, _W)
    xt = x.reshape(_N, _NCHUNK, _C).transpose(1, 0, 2).reshape(_NCHUNK * _N, _C)
    agg, cnt = _sc_call(xt, src, dst, typ)
    aggv = agg.reshape(_R, _N, _D)
    cntv = cnt.reshape(_NCORE, _R, _N, _C)
    return _tc_call(x, aggv, cntv, weight, w_comp, loop_weight,
                    h_bias.reshape(1, _D))


# speedup vs baseline: 16.0092x; 1.1497x over previous
import jax
import jax.numpy as jnp
from jax import lax
from jax.experimental import pallas as pl
from jax.experimental.pallas import tpu as pltpu
from jax.experimental.pallas import tpu_sc as plsc

_N = 10000
_E = 320000
_R = 8
_B = 4
_D = 128
_C = 16
_NCHUNK = _D // _C
_RN = _R * _N

_NCORE = 2
_NSUB = 16
_PASSES = _NCHUNK // _NCORE
_W = 128
_NWIN = 160
_EPAD = _NSUB * _NWIN * _W
_NB = 4
_RPT = _RN // _NSUB
_ZROWS = 40


def _sc_body(xt, srch, dsth, typh, aggo, cnto,
             agg_s, buf_r, gidx, gbs, zb3, sems, ssems):
    core = lax.axis_index("c")
    sub = lax.axis_index("s")

    pltpu.sync_copy(dsth.at[sub], buf_r)
    pltpu.sync_copy(typh.at[sub], gidx)

    def _mk_ridx(w, carry):
        for j in range(_W // 16):
            t16 = gidx[w, pl.ds(j * 16, 16)]
            d16 = buf_r[w, pl.ds(j * 16, 16)]
            buf_r[w, pl.ds(j * 16, 16)] = t16 * _N + d16
        return carry
    lax.fori_loop(0, _NWIN, _mk_ridx, 0)
    pltpu.sync_copy(srch.at[sub], gidx)

    z16 = jnp.zeros((16,), jnp.float32)

    def _fill_zb3(i, carry):
        zb3[i, :] = z16
        return carry
    lax.fori_loop(0, _ZROWS, _fill_zb3, 0)

    one16 = jnp.ones((16,), jnp.float32)

    def _fill_ones(i, carry):
        gbs[0, i, :] = one16
        return carry
    lax.fori_loop(0, _W, _fill_ones, 0)

    def _zero_agg():
        def _z(z, c2):
            pltpu.sync_copy(zb3, agg_s.at[pl.ds(sub * _RPT + z * _ZROWS, _ZROWS)])
            return c2
        lax.fori_loop(0, _RPT // _ZROWS, _z, 0)

    _zero_agg()
    plsc.subcore_barrier()
    halfw = _NWIN // 2
    w0 = core * halfw

    def _cnt_win(w, carry):
        pltpu.sync_copy(gbs.at[0], agg_s.at[buf_r.at[w0 + w]], add=True)
        return carry
    lax.fori_loop(0, halfw, _cnt_win, 0)
    plsc.subcore_barrier()
    pltpu.sync_copy(agg_s.at[pl.ds(sub * _RPT, _RPT)],
                    cnto.at[core, pl.ds(sub * _RPT, _RPT), :])

    for p in range(_PASSES):
        if p == 0:
            def _mk_gidx(w, carry):
                for j in range(_W // 16):
                    s16 = gidx[w, pl.ds(j * 16, 16)]
                    gidx[w, pl.ds(j * 16, 16)] = s16 * _NCHUNK + core * _PASSES
                return carry
        else:
            def _mk_gidx(w, carry):
                for j in range(_W // 16):
                    s16 = gidx[w, pl.ds(j * 16, 16)]
                    gidx[w, pl.ds(j * 16, 16)] = s16 + 1
                return carry
        lax.fori_loop(0, _NWIN, _mk_gidx, 0)

        _zero_agg()
        plsc.subcore_barrier()

        def _group(i, carry):
            wbase = i * _NB
            descs = [
                pltpu.async_copy(xt.at[gidx.at[wbase + b]], gbs.at[b], sems.at[b])
                for b in range(_NB)
            ]
            sdescs = []
            for b in range(_NB):
                descs[b].wait()
                sdescs.append(pltpu.async_copy(
                    gbs.at[b], agg_s.at[buf_r.at[wbase + b]], ssems.at[b],
                    add=True))
            for d in sdescs:
                d.wait()
            return carry
        lax.fori_loop(0, _NWIN // _NB, _group, 0)
        plsc.subcore_barrier()

        pltpu.sync_copy(
            agg_s.at[pl.ds(sub * _RPT, _RPT)],
            aggo.at[pl.ds(sub * _RPT, _RPT), pl.ds((core * _PASSES + p) * _C, _C)])


_sc_call = pl.kernel(
    _sc_body,
    out_type=(
        jax.ShapeDtypeStruct((_RN, _D), jnp.float32),
        jax.ShapeDtypeStruct((_NCORE, _RN, _C), jnp.float32),
    ),
    mesh=plsc.VectorSubcoreMesh(core_axis_name="c", subcore_axis_name="s"),
    compiler_params=pltpu.CompilerParams(use_tc_tiling_on_sc=False),
    scratch_types=[
        pltpu.VMEM_SHARED((_RN + _W, _C), jnp.float32),
        pltpu.VMEM((_NWIN, _W), jnp.int32),
        pltpu.VMEM((_NWIN, _W), jnp.int32),
        pltpu.VMEM((_NB, _W, _C), jnp.float32),
        pltpu.VMEM((_ZROWS, _C), jnp.float32),
        pltpu.SemaphoreType.DMA((_NB,)),
        pltpu.SemaphoreType.DMA((_NB,)),
    ],
)

_BN = 1000


def _tc_body(x_ref, agg_ref, cnt_ref, w_ref, wc_ref, lw_ref, b_ref, o_ref):
    cnt = cnt_ref[0, :, :, 0] + cnt_ref[1, :, :, 0]
    recip = 1.0 / jnp.maximum(cnt, 1.0)
    acc = jnp.dot(x_ref[...], lw_ref[...], preferred_element_type=jnp.float32)
    g = [None] * _B
    for r in range(_R):
        a = agg_ref[r] * recip[r][:, None]
        for b in range(_B):
            contrib = wc_ref[r, b] * a
            g[b] = contrib if g[b] is None else g[b] + contrib
    for b in range(_B):
        acc = acc + jnp.dot(g[b], w_ref[b], preferred_element_type=jnp.float32)
    o_ref[...] = acc + b_ref[...]


_tc_call = pl.pallas_call(
    _tc_body,
    grid=(_N // _BN,),
    in_specs=[
        pl.BlockSpec((_BN, _D), lambda i: (i, 0)),
        pl.BlockSpec((_R, _BN, _D), lambda i: (0, i, 0)),
        pl.BlockSpec((_NCORE, _R, _BN, _C), lambda i: (0, 0, i, 0)),
        pl.BlockSpec((_B, _D, _D), lambda i: (0, 0, 0)),
        pl.BlockSpec(memory_space=pltpu.SMEM),
        pl.BlockSpec((_D, _D), lambda i: (0, 0)),
        pl.BlockSpec((1, _D), lambda i: (0, 0)),
    ],
    out_specs=pl.BlockSpec((_BN, _D), lambda i: (i, 0)),
    out_shape=jax.ShapeDtypeStruct((_N, _D), jnp.float32),
)


def kernel(x, edge_index, edge_type, weight, w_comp, loop_weight, h_bias):
    npad = _EPAD - _E
    pi = jnp.arange(npad, dtype=jnp.int32)
    src = jnp.concatenate([edge_index[0], pi % _N]).reshape(_NSUB, _NWIN, _W)
    dst = jnp.concatenate([edge_index[1], pi % _W]).reshape(_NSUB, _NWIN, _W)
    typ = jnp.concatenate(
        [edge_type, jnp.full((npad,), _R, jnp.int32)]).reshape(_NSUB, _NWIN, _W)
    xt = x.reshape(_N * _NCHUNK, _C)
    agg, cnt = _sc_call(xt, src, dst, typ)
    aggv = agg.reshape(_R, _N, _D)
    cntv = cnt.reshape(_NCORE, _R, _N, _C)
    return _tc_call(x, aggv, cntv, weight, w_comp, loop_weight,
                    h_bias.reshape(1, _D))
